# baseline (speedup 1.0000x reference)
import functools

import jax
from jax import lax
import jax.numpy as jnp
from jax.experimental import pallas as pl
from jax.experimental.pallas import tpu as pltpu
from jax.experimental.pallas import tpu_sc as plsc

_NUM_ROWS = 1000
_D = 4096
_B = 16384
_NC = 2
_NS = 16
_NW = _NC * _NS
_SPLIT = 2
_DS = _D // _SPLIT
_BS = _B * _SPLIT
_BPW = _BS // _NW
_C = 8
_NCHUNK = _BPW // _C
_NBUF = 6
_GAHEAD = 4


def kernel(indices, weight):
    flat = weight.reshape(_NUM_ROWS * _SPLIT, _DS)
    idx = indices.astype(jnp.int32)
    if _SPLIT > 1:
        idx = (
            idx[:, None] * _SPLIT + jnp.arange(_SPLIT, dtype=jnp.int32)
        ).reshape(_BS)
    mesh = plsc.VectorSubcoreMesh(
        core_axis_name="core", subcore_axis_name="subcore"
    )

    scratch = (
        [pltpu.VMEM((_BPW,), jnp.int32)]
        + [pltpu.VMEM((_C, _DS), jnp.float32) for _ in range(_NBUF)]
        + [pltpu.SemaphoreType.DMA for _ in range(2 * _NBUF)]
    )

    @functools.partial(
        pl.kernel,
        out_type=jax.ShapeDtypeStruct((_BS, _DS), jnp.float32),
        mesh=mesh,
        scratch_types=scratch,
    )
    def gather_kernel(x_hbm, i_hbm, o_hbm, idx_v, *rest):
        bufs = rest[:_NBUF]
        gsems = rest[_NBUF:2 * _NBUF]
        ssems = rest[2 * _NBUF:]

        wid = lax.axis_index("subcore") * _NC + lax.axis_index("core")
        base = wid * _BPW
        pltpu.sync_copy(i_hbm.at[pl.ds(base, _BPW)], idx_v)

        def gather_copy(g, j):
            return pltpu.make_async_copy(
                x_hbm.at[idx_v.at[pl.ds(g * _C, _C)]], bufs[j], gsems[j]
            )

        def store_copy(g, j):
            return pltpu.make_async_copy(
                bufs[j], o_hbm.at[pl.ds(base + g * _C, _C)], ssems[j]
            )

        for g in range(_GAHEAD):
            gather_copy(g, g).start()

        @pl.loop(0, _NCHUNK + (-_NCHUNK) % _NBUF, step=_NBUF)
        def _(g0):
            for b in range(_NBUF):
                g = g0 + b
                jn = (b + _GAHEAD) % _NBUF

                @pl.when(g < _NCHUNK)
                def _():
                    @pl.when(g + _GAHEAD < _NCHUNK)
                    def _():
                        @pl.when(g >= _NBUF - _GAHEAD)
                        def _():
                            store_copy(g - (_NBUF - _GAHEAD), jn).wait()

                        gather_copy(g + _GAHEAD, jn).start()

                    gather_copy(g, b).wait()
                    store_copy(g, b).start()

        for g in range(_NCHUNK - _NBUF, _NCHUNK):
            store_copy(g, g % _NBUF).wait()

    out = gather_kernel(flat, idx)
    return out.reshape(_B, 64, 64)


# speedup vs baseline: 2.0233x; 2.0233x over previous
import functools

import jax
from jax import lax
import jax.numpy as jnp
from jax.experimental import pallas as pl
from jax.experimental.pallas import tpu as pltpu
from jax.experimental.pallas import tpu_sc as plsc

_NUM_ROWS = 1000
_D = 4096
_B = 16384
_NC = 2
_NS = 16
_NW = _NC * _NS
_SPLIT = 1
_DS = _D // _SPLIT
_BS = _B * _SPLIT
_BPW = _BS // _NW
_C = 8
_NCHUNK = _BPW // _C
_NBUF = 3
_GAHEAD = 2


def kernel(indices, weight):
    flat = weight.reshape(_NUM_ROWS * _SPLIT, _DS)
    idx = indices.astype(jnp.int32)
    if _SPLIT > 1:
        idx = (
            idx[:, None] * _SPLIT + jnp.arange(_SPLIT, dtype=jnp.int32)
        ).reshape(_BS)
    mesh = plsc.VectorSubcoreMesh(
        core_axis_name="core", subcore_axis_name="subcore"
    )

    scratch = (
        [pltpu.VMEM((_BPW,), jnp.int32)]
        + [pltpu.VMEM((_C, _DS), jnp.float32) for _ in range(_NBUF)]
        + [pltpu.SemaphoreType.DMA for _ in range(2 * _NBUF)]
    )

    @functools.partial(
        pl.kernel,
        out_type=jax.ShapeDtypeStruct((_BS, _DS), jnp.float32),
        mesh=mesh,
        scratch_types=scratch,
    )
    def gather_kernel(x_hbm, i_hbm, o_hbm, idx_v, *rest):
        bufs = rest[:_NBUF]
        gsems = rest[_NBUF:2 * _NBUF]
        ssems = rest[2 * _NBUF:]

        wid = lax.axis_index("subcore") * _NC + lax.axis_index("core")
        base = wid * _BPW
        pltpu.sync_copy(i_hbm.at[pl.ds(base, _BPW)], idx_v)

        def gather_copy(g, j):
            return pltpu.make_async_copy(
                x_hbm.at[idx_v.at[pl.ds(g * _C, _C)]], bufs[j], gsems[j]
            )

        def store_copy(g, j):
            return pltpu.make_async_copy(
                bufs[j], o_hbm.at[pl.ds(base + g * _C, _C)], ssems[j]
            )

        for g in range(_GAHEAD):
            gather_copy(g, g).start()

        @pl.loop(0, _NCHUNK + (-_NCHUNK) % _NBUF, step=_NBUF)
        def _(g0):
            for b in range(_NBUF):
                g = g0 + b
                jn = (b + _GAHEAD) % _NBUF

                @pl.when(g < _NCHUNK)
                def _():
                    @pl.when(g + _GAHEAD < _NCHUNK)
                    def _():
                        @pl.when(g >= _NBUF - _GAHEAD)
                        def _():
                            store_copy(g - (_NBUF - _GAHEAD), jn).wait()

                        gather_copy(g + _GAHEAD, jn).start()

                    gather_copy(g, b).wait()
                    store_copy(g, b).start()

        for g in range(_NCHUNK - _NBUF, _NCHUNK):
            store_copy(g, g % _NBUF).wait()

    out = gather_kernel(flat, idx)
    return out.reshape(_B, 64, 64)


# speedup vs baseline: 2.0242x; 1.0005x over previous
import functools

import jax
from jax import lax
import jax.numpy as jnp
from jax.experimental import pallas as pl
from jax.experimental.pallas import tpu as pltpu
from jax.experimental.pallas import tpu_sc as plsc

_NUM_ROWS = 1000
_D = 4096
_B = 16384
_NC = 2
_NS = 16
_NW = _NC * _NS
_SPLIT = 1
_DS = _D // _SPLIT
_BS = _B * _SPLIT
_BPW = _BS // _NW
_C = 8
_NCHUNK = _BPW // _C
_NBUF = 3
_GAHEAD = 1


def kernel(indices, weight):
    flat = weight.reshape(_NUM_ROWS * _SPLIT, _DS)
    idx = indices.astype(jnp.int32)
    if _SPLIT > 1:
        idx = (
            idx[:, None] * _SPLIT + jnp.arange(_SPLIT, dtype=jnp.int32)
        ).reshape(_BS)
    mesh = plsc.VectorSubcoreMesh(
        core_axis_name="core", subcore_axis_name="subcore"
    )

    scratch = (
        [pltpu.VMEM((_BPW,), jnp.int32)]
        + [pltpu.VMEM((_C, _DS), jnp.float32) for _ in range(_NBUF)]
        + [pltpu.SemaphoreType.DMA for _ in range(2 * _NBUF)]
    )

    @functools.partial(
        pl.kernel,
        out_type=jax.ShapeDtypeStruct((_BS, _DS), jnp.float32),
        mesh=mesh,
        scratch_types=scratch,
    )
    def gather_kernel(x_hbm, i_hbm, o_hbm, idx_v, *rest):
        bufs = rest[:_NBUF]
        gsems = rest[_NBUF:2 * _NBUF]
        ssems = rest[2 * _NBUF:]

        wid = lax.axis_index("subcore") * _NC + lax.axis_index("core")
        base = wid * _BPW
        pltpu.sync_copy(i_hbm.at[pl.ds(base, _BPW)], idx_v)

        def gather_copy(g, j):
            return pltpu.make_async_copy(
                x_hbm.at[idx_v.at[pl.ds(g * _C, _C)]], bufs[j], gsems[j]
            )

        def store_copy(g, j):
            return pltpu.make_async_copy(
                bufs[j], o_hbm.at[pl.ds(base + g * _C, _C)], ssems[j]
            )

        for g in range(_GAHEAD):
            gather_copy(g, g).start()

        @pl.loop(0, _NCHUNK + (-_NCHUNK) % _NBUF, step=_NBUF)
        def _(g0):
            for b in range(_NBUF):
                g = g0 + b
                jn = (b + _GAHEAD) % _NBUF

                @pl.when(g < _NCHUNK)
                def _():
                    @pl.when(g + _GAHEAD < _NCHUNK)
                    def _():
                        @pl.when(g >= _NBUF - _GAHEAD)
                        def _():
                            store_copy(g - (_NBUF - _GAHEAD), jn).wait()

                        gather_copy(g + _GAHEAD, jn).start()

                    gather_copy(g, b).wait()
                    store_copy(g, b).start()

        for g in range(_NCHUNK - _NBUF, _NCHUNK):
            store_copy(g, g % _NBUF).wait()

    out = gather_kernel(flat, idx)
    return out.reshape(_B, 64, 64)
